# baseline (speedup 1.0000x reference)
import jax
import jax.numpy as jnp
from jax.experimental import pallas as pl
from jax.experimental.pallas import tpu as pltpu

B, P, L, C = 16, 1024, 32, 9
H = 64
OUT = 64
N = B * P

TILE = 128


def _relu(x):
    return jnp.maximum(x, 0.0)


def _fused_body(x_ref, m_ref, pW1_ref, pb1_ref, pW2_ref, pb2_ref, pW3_ref,
                pb3_ref, mW1_ref, mb1_ref, mW2_ref, mb2_ref, mW3_ref,
                mb3_ref, oW1_ref, ob1_ref, oW2_ref, ob2_ref, out_ref):
    f32 = jnp.float32
    x = x_ref[...]
    m = m_ref[...]

    h = _relu(jnp.dot(x, pW1_ref[...], preferred_element_type=f32)
              + pb1_ref[...])
    h = _relu(jnp.dot(h, pW2_ref[...], preferred_element_type=f32)
              + pb2_ref[...])
    h = jnp.dot(h, pW3_ref[...], preferred_element_type=f32) + pb3_ref[...]
    hm = h * m

    pooled = jnp.max(hm.reshape(TILE, L, H), axis=1)

    pc = jnp.dot(pooled, mW1_ref[H:, :], preferred_element_type=f32)
    pcb = jnp.broadcast_to(pc[:, None, :], (TILE, L, H)).reshape(TILE * L, H)
    g = _relu(jnp.dot(hm, mW1_ref[:H, :], preferred_element_type=f32)
              + pcb + mb1_ref[...])
    g = _relu(jnp.dot(g, mW2_ref[...], preferred_element_type=f32)
              + mb2_ref[...])
    g = jnp.dot(g, mW3_ref[...], preferred_element_type=f32) + mb3_ref[...]
    gm = g * m

    fb = jnp.max(gm.reshape(TILE, L, H), axis=1)
    vm = jnp.max(m.reshape(TILE, L, 1), axis=1)

    o = _relu(jnp.dot(fb, oW1_ref[...], preferred_element_type=f32)
              + ob1_ref[...])
    o = jnp.dot(o, oW2_ref[...], preferred_element_type=f32) + ob2_ref[...]
    out_ref[...] = o * vm


def kernel(polylines, polylines_mask, pW1, pb1, pW2, pb2, pW3, pb3,
           mW1, mb1, mW2, mb2, mW3, mb3, oW1, ob1, oW2, ob2):
    x = polylines.reshape(N * L, C)
    m = polylines_mask.reshape(N * L, 1).astype(jnp.float32)

    row = lambda b: b.reshape(1, -1)
    full = lambda s: pl.BlockSpec(s, lambda i: (0, 0))

    out = pl.pallas_call(
        _fused_body,
        grid=(N // TILE,),
        in_specs=[
            pl.BlockSpec((TILE * L, C), lambda i: (i, 0)),
            pl.BlockSpec((TILE * L, 1), lambda i: (i, 0)),
            full((C, H)), full((1, H)),
            full((H, H)), full((1, H)),
            full((H, H)), full((1, H)),
            full((2 * H, H)), full((1, H)),
            full((H, H)), full((1, H)),
            full((H, H)), full((1, H)),
            full((H, H)), full((1, H)),
            full((H, OUT)), full((1, OUT)),
        ],
        out_specs=pl.BlockSpec((TILE, OUT), lambda i: (i, 0)),
        out_shape=jax.ShapeDtypeStruct((N, OUT), jnp.float32),
        compiler_params=pltpu.CompilerParams(
            dimension_semantics=("parallel",)),
    )(x, m, pW1, row(pb1), pW2, row(pb2), pW3, row(pb3),
      mW1, row(mb1), mW2, row(mb2), mW3, row(mb3),
      oW1, row(ob1), oW2, row(ob2))
    return out.reshape(B, P, OUT)


# speedup vs baseline: 1.2642x; 1.2642x over previous
import jax
import jax.numpy as jnp
from jax.experimental import pallas as pl
from jax.experimental.pallas import tpu as pltpu

B, P, L, C = 16, 1024, 32, 9
H = 64
OUT = 64
N = B * P

TILE = 128


def _relu(x):
    return jnp.maximum(x, 0.0)


def _fused_body(x_ref, m_ref, pW1_ref, pb1_ref, pW2_ref, pb2_ref, pW3_ref,
                pb3_ref, mW1_ref, mb1_ref, mW2_ref, mb2_ref, mW3_ref,
                mb3_ref, oW1_ref, ob1_ref, oW2_ref, ob2_ref, out_ref):
    f32 = jnp.float32
    x = x_ref[...].reshape(L * TILE, C)
    m = m_ref[...].reshape(L * TILE, 1)

    h = _relu(jnp.dot(x, pW1_ref[...], preferred_element_type=f32)
              + pb1_ref[...])
    h = _relu(jnp.dot(h, pW2_ref[...], preferred_element_type=f32)
              + pb2_ref[...])
    h = jnp.dot(h, pW3_ref[...], preferred_element_type=f32) + pb3_ref[...]
    hm = h * m

    pooled = jnp.max(hm.reshape(L, TILE, H), axis=0)

    pc = jnp.dot(pooled, mW1_ref[H:, :], preferred_element_type=f32)
    pcb = jnp.broadcast_to(pc[None, :, :], (L, TILE, H)).reshape(L * TILE, H)
    g = _relu(jnp.dot(hm, mW1_ref[:H, :], preferred_element_type=f32)
              + pcb + mb1_ref[...])
    g = _relu(jnp.dot(g, mW2_ref[...], preferred_element_type=f32)
              + mb2_ref[...])
    g = jnp.dot(g, mW3_ref[...], preferred_element_type=f32) + mb3_ref[...]
    gm = g * m

    fb = jnp.max(gm.reshape(L, TILE, H), axis=0)
    vm = jnp.max(m.reshape(L, TILE, 1), axis=0)

    o = _relu(jnp.dot(fb, oW1_ref[...], preferred_element_type=f32)
              + ob1_ref[...])
    o = jnp.dot(o, oW2_ref[...], preferred_element_type=f32) + ob2_ref[...]
    out_ref[...] = o * vm


def kernel(polylines, polylines_mask, pW1, pb1, pW2, pb2, pW3, pb3,
           mW1, mb1, mW2, mb2, mW3, mb3, oW1, ob1, oW2, ob2):
    x = polylines.reshape(N, L, C).transpose(1, 0, 2)
    m = polylines_mask.reshape(N, L).T[:, :, None].astype(jnp.float32)

    row = lambda b: b.reshape(1, -1)
    full = lambda s: pl.BlockSpec(s, lambda i: (0, 0))

    out = pl.pallas_call(
        _fused_body,
        grid=(N // TILE,),
        in_specs=[
            pl.BlockSpec((L, TILE, C), lambda i: (0, i, 0)),
            pl.BlockSpec((L, TILE, 1), lambda i: (0, i, 0)),
            full((C, H)), full((1, H)),
            full((H, H)), full((1, H)),
            full((H, H)), full((1, H)),
            full((2 * H, H)), full((1, H)),
            full((H, H)), full((1, H)),
            full((H, H)), full((1, H)),
            full((H, H)), full((1, H)),
            full((H, OUT)), full((1, OUT)),
        ],
        out_specs=pl.BlockSpec((TILE, OUT), lambda i: (i, 0)),
        out_shape=jax.ShapeDtypeStruct((N, OUT), jnp.float32),
        compiler_params=pltpu.CompilerParams(
            dimension_semantics=("parallel",)),
    )(x, m, pW1, row(pb1), pW2, row(pb2), pW3, row(pb3),
      mW1, row(mb1), mW2, row(mb2), mW3, row(mb3),
      oW1, row(ob1), oW2, row(ob2))
    return out.reshape(B, P, OUT)


# speedup vs baseline: 1.2779x; 1.0109x over previous
import jax
import jax.numpy as jnp
from jax.experimental import pallas as pl
from jax.experimental.pallas import tpu as pltpu

B, P, L, C = 16, 1024, 32, 9
H = 64
OUT = 64
N = B * P

TILE = 128


def _relu(x):
    return jnp.maximum(x, 0.0)


def _fused_body(x_ref, m_ref, pW1_ref, pb1_ref, pW2_ref, pb2_ref, pW3_ref,
                pb3_ref, mW1_ref, mb1_ref, mW2_ref, mb2_ref, mW3_ref,
                mb3_ref, oW1_ref, ob1_ref, oW2_ref, ob2_ref, out_ref):
    f32 = jnp.float32
    bf = jnp.bfloat16
    x = x_ref[...].reshape(L * TILE, C)
    m = m_ref[...].reshape(L * TILE, 1)

    h = _relu(jnp.dot(x, pW1_ref[...], preferred_element_type=f32)
              + pb1_ref[...])
    h = _relu(jnp.dot(h.astype(bf), pW2_ref[...], preferred_element_type=f32)
              + pb2_ref[...])
    h = (jnp.dot(h.astype(bf), pW3_ref[...], preferred_element_type=f32)
         + pb3_ref[...])
    hm = h * m

    pooled = jnp.max(hm.reshape(L, TILE, H), axis=0)

    hmb = hm.astype(bf)
    pc = jnp.dot(pooled.astype(bf), mW1_ref[H:, :],
                 preferred_element_type=f32)
    pcb = jnp.broadcast_to(pc[None, :, :], (L, TILE, H)).reshape(L * TILE, H)
    g = _relu(jnp.dot(hmb, mW1_ref[:H, :], preferred_element_type=f32)
              + pcb + mb1_ref[...])
    g = _relu(jnp.dot(g.astype(bf), mW2_ref[...], preferred_element_type=f32)
              + mb2_ref[...])
    g = (jnp.dot(g.astype(bf), mW3_ref[...], preferred_element_type=f32)
         + mb3_ref[...])
    gm = g * m

    fb = jnp.max(gm.reshape(L, TILE, H), axis=0)
    vm = jnp.max(m.reshape(L, TILE, 1), axis=0)

    o = _relu(jnp.dot(fb.astype(bf), oW1_ref[...],
                      preferred_element_type=f32) + ob1_ref[...])
    o = (jnp.dot(o.astype(bf), oW2_ref[...], preferred_element_type=f32)
         + ob2_ref[...])
    out_ref[...] = o * vm


def kernel(polylines, polylines_mask, pW1, pb1, pW2, pb2, pW3, pb3,
           mW1, mb1, mW2, mb2, mW3, mb3, oW1, ob1, oW2, ob2):
    bf = jnp.bfloat16
    x = polylines.astype(bf).reshape(N, L, C).transpose(1, 0, 2)
    m = polylines_mask.reshape(N, L).T[:, :, None].astype(jnp.float32)
    pW1, pW2, pW3 = pW1.astype(bf), pW2.astype(bf), pW3.astype(bf)
    mW1, mW2, mW3 = mW1.astype(bf), mW2.astype(bf), mW3.astype(bf)
    oW1, oW2 = oW1.astype(bf), oW2.astype(bf)

    row = lambda b: b.reshape(1, -1)
    full = lambda s: pl.BlockSpec(s, lambda i: (0, 0))

    out = pl.pallas_call(
        _fused_body,
        grid=(N // TILE,),
        in_specs=[
            pl.BlockSpec((L, TILE, C), lambda i: (0, i, 0)),
            pl.BlockSpec((L, TILE, 1), lambda i: (0, i, 0)),
            full((C, H)), full((1, H)),
            full((H, H)), full((1, H)),
            full((H, H)), full((1, H)),
            full((2 * H, H)), full((1, H)),
            full((H, H)), full((1, H)),
            full((H, H)), full((1, H)),
            full((H, H)), full((1, H)),
            full((H, OUT)), full((1, OUT)),
        ],
        out_specs=pl.BlockSpec((TILE, OUT), lambda i: (i, 0)),
        out_shape=jax.ShapeDtypeStruct((N, OUT), jnp.float32),
        compiler_params=pltpu.CompilerParams(
            dimension_semantics=("parallel",)),
    )(x, m, pW1, row(pb1), pW2, row(pb2), pW3, row(pb3),
      mW1, row(mb1), mW2, row(mb2), mW3, row(mb3),
      oW1, row(ob1), oW2, row(ob2))
    return out.reshape(B, P, OUT)


# speedup vs baseline: 1.5161x; 1.1864x over previous
import jax
import jax.numpy as jnp
from jax.experimental import pallas as pl
from jax.experimental.pallas import tpu as pltpu

B, P, L, C = 16, 1024, 32, 9
H = 64
OUT = 64
N = B * P
L2 = L // 2
C2 = 2 * C
H2 = 2 * H

TILE = 256
RW = L2 * TILE


def _relu(x):
    return jnp.maximum(x, 0.0)


def _fused_body(x_ref, m_ref, pW1_ref, pb1_ref, pW2_ref, pb2_ref, pW3_ref,
                pb3_ref, mW1a_ref, mW1b_ref, mb1_ref, mW2_ref, mb2_ref,
                mW3_ref, mb3_ref, oW1_ref, ob1_ref, oW2_ref, ob2_ref,
                out_ref):
    f32 = jnp.float32
    bf = jnp.bfloat16
    x = x_ref[...].reshape(RW, C2)
    m = m_ref[...]

    mp = jnp.concatenate(
        [jnp.broadcast_to(m[:L2], (L2, TILE, H)),
         jnp.broadcast_to(m[L2:], (L2, TILE, H))], axis=-1).reshape(RW, H2)

    h = _relu(jnp.dot(x, pW1_ref[...], preferred_element_type=f32)
              + pb1_ref[...])
    h = _relu(jnp.dot(h.astype(bf), pW2_ref[...], preferred_element_type=f32)
              + pb2_ref[...])
    h = (jnp.dot(h.astype(bf), pW3_ref[...], preferred_element_type=f32)
         + pb3_ref[...])
    hm = h * mp

    pooled2 = jnp.max(hm.reshape(L2, TILE, H2), axis=0)
    pooled = jnp.maximum(pooled2[:, :H], pooled2[:, H:])

    pc = jnp.dot(pooled.astype(bf), mW1b_ref[...],
                 preferred_element_type=f32)
    pc2 = jnp.concatenate([pc, pc], axis=-1)
    pcb = jnp.broadcast_to(pc2[None], (L2, TILE, H2)).reshape(RW, H2)
    g = _relu(jnp.dot(hm.astype(bf), mW1a_ref[...],
                      preferred_element_type=f32) + pcb + mb1_ref[...])
    g = _relu(jnp.dot(g.astype(bf), mW2_ref[...], preferred_element_type=f32)
              + mb2_ref[...])
    g = (jnp.dot(g.astype(bf), mW3_ref[...], preferred_element_type=f32)
         + mb3_ref[...])
    gm = g * mp

    fb2 = jnp.max(gm.reshape(L2, TILE, H2), axis=0)
    fb = jnp.maximum(fb2[:, :H], fb2[:, H:])
    vm = jnp.max(m, axis=0)

    o = _relu(jnp.dot(fb.astype(bf), oW1_ref[...],
                      preferred_element_type=f32) + ob1_ref[...])
    o = (jnp.dot(o.astype(bf), oW2_ref[...], preferred_element_type=f32)
         + ob2_ref[...])
    out_ref[...] = o * vm


def _bdiag(W):
    k, n = W.shape
    z = jnp.zeros((k, n), W.dtype)
    return jnp.concatenate(
        [jnp.concatenate([W, z], axis=1),
         jnp.concatenate([z, W], axis=1)], axis=0)


def kernel(polylines, polylines_mask, pW1, pb1, pW2, pb2, pW3, pb3,
           mW1, mb1, mW2, mb2, mW3, mb3, oW1, ob1, oW2, ob2):
    bf = jnp.bfloat16
    x = (polylines.astype(bf).reshape(N, 2, L2, C)
         .transpose(2, 0, 1, 3).reshape(L2, N, C2))
    m = polylines_mask.reshape(N, L).T[:, :, None].astype(jnp.float32)

    pW1d, pW2d, pW3d = _bdiag(pW1.astype(bf)), _bdiag(pW2.astype(bf)), \
        _bdiag(pW3.astype(bf))
    mW1a, mW1b = _bdiag(mW1[:H].astype(bf)), mW1[H:].astype(bf)
    mW2d, mW3d = _bdiag(mW2.astype(bf)), _bdiag(mW3.astype(bf))
    oW1b, oW2b = oW1.astype(bf), oW2.astype(bf)
    two = lambda b: jnp.concatenate([b, b]).reshape(1, H2)
    row = lambda b: b.reshape(1, -1)
    full = lambda s: pl.BlockSpec(s, lambda i: (0, 0))

    out = pl.pallas_call(
        _fused_body,
        grid=(N // TILE,),
        in_specs=[
            pl.BlockSpec((L2, TILE, C2), lambda i: (0, i, 0)),
            pl.BlockSpec((L, TILE, 1), lambda i: (0, i, 0)),
            full((C2, H2)), full((1, H2)),
            full((H2, H2)), full((1, H2)),
            full((H2, H2)), full((1, H2)),
            full((H2, H2)), full((H, H)), full((1, H2)),
            full((H2, H2)), full((1, H2)),
            full((H2, H2)), full((1, H2)),
            full((H, H)), full((1, H)),
            full((H, OUT)), full((1, OUT)),
        ],
        out_specs=pl.BlockSpec((TILE, OUT), lambda i: (i, 0)),
        out_shape=jax.ShapeDtypeStruct((N, OUT), jnp.float32),
        compiler_params=pltpu.CompilerParams(
            dimension_semantics=("parallel",)),
    )(x, m, pW1d, two(pb1), pW2d, two(pb2), pW3d, two(pb3),
      mW1a, mW1b, two(mb1), mW2d, two(mb2), mW3d, two(mb3),
      oW1b, row(ob1), oW2b, row(ob2))
    return out.reshape(B, P, OUT)
